# Initial kernel scaffold; baseline (speedup 1.0000x reference)
import jax
import jax.numpy as jnp
from jax.experimental import pallas as pl


def kernel(x, edge_index, edge_attr, batch, params):
    raise NotImplementedError("write your pallas kernel here")


# speedup vs baseline: 1.0724x; 1.0724x over previous
import jax
import jax.numpy as jnp
from jax.experimental import pallas as pl


def _mm_body(x_ref, wl_ref, wr_ref, xl_ref, xr_ref):
    x = x_ref[...]
    xl_ref[...] = jnp.dot(x, wl_ref[...], preferred_element_type=jnp.float32)
    xr_ref[...] = jnp.dot(x, wr_ref[...], preferred_element_type=jnp.float32)


def _mm(x, wl, wr):
    n, d = x.shape
    hc = wl.shape[1]
    br = 400
    return pl.pallas_call(
        _mm_body,
        grid=(n // br,),
        in_specs=[
            pl.BlockSpec((br, d), lambda i: (i, 0)),
            pl.BlockSpec((d, hc), lambda i: (0, 0)),
            pl.BlockSpec((d, hc), lambda i: (0, 0)),
        ],
        out_specs=[
            pl.BlockSpec((br, hc), lambda i: (i, 0)),
            pl.BlockSpec((br, hc), lambda i: (i, 0)),
        ],
        out_shape=[jax.ShapeDtypeStruct((n, hc), jnp.float32)] * 2,
    )(x, wl, wr)


def kernel(x, edge_index, edge_attr, batch, params):
    src = edge_index[0].astype(jnp.int32)
    dst = edge_index[1].astype(jnp.int32)
    a = edge_attr[:, 0]
    n = x.shape[0]
    for i, p in enumerate(params):
        h, c = p["att"].shape
        hc = h * c
        xl, xr = _mm(x, p["Wl"], p["Wr"])
        m = (jnp.take(xl, src, axis=0) + jnp.take(xr, dst, axis=0)
             + a[:, None] * p["We"][0][None, :]).reshape(-1, h, c)
        m = jnp.where(m > 0, m, 0.2 * m)
        logits = jnp.sum(m * p["att"][None], axis=-1)
        ex = jnp.exp(logits)
        den = jax.ops.segment_sum(ex, dst, num_segments=n)
        alpha = ex / (jnp.take(den, dst, axis=0) + 1e-16)
        msg = jnp.take(xl, src, axis=0).reshape(-1, h, c) * alpha[..., None]
        out = jax.ops.segment_sum(msg, dst, num_segments=n).reshape(-1, hc) + p["b"]
        mu = jnp.mean(out, -1, keepdims=True)
        var = jnp.var(out, -1, keepdims=True)
        out = (out - mu) / jnp.sqrt(var + 1e-5) * p["gamma"] + p["beta"]
        if i < 2:
            out = jax.nn.gelu(out)
        x = x + out
    return x[None]


# speedup vs baseline: 9.3168x; 8.6879x over previous
import functools

import jax
import jax.numpy as jnp
from jax import lax
from jax.experimental import pallas as pl
from jax.experimental.pallas import tpu as pltpu
from jax.experimental.pallas import tpu_sc as plsc

N = 10000
E = 320000
D = 128
NC = 2
NS = 16
NW = NC * NS
EPW = E // NW
N2 = 10240
ROWS_PT = N2 // NS

_MESH = plsc.VectorSubcoreMesh(core_axis_name="c", subcore_axis_name="s")


def _iota16():
    return lax.broadcasted_iota(jnp.int32, (16,), 0)


def _mm_body(x_ref, wl_ref, wr_ref, xl_ref, xr_ref):
    x = x_ref[...]
    xl_ref[...] = jnp.dot(x, wl_ref[...], preferred_element_type=jnp.float32)
    xr_ref[...] = jnp.dot(x, wr_ref[...], preferred_element_type=jnp.float32)


def _mm(x, wl, wr):
    br = 400
    return pl.pallas_call(
        _mm_body,
        grid=(N // br,),
        in_specs=[
            pl.BlockSpec((br, D), lambda i: (i, 0)),
            pl.BlockSpec((D, D), lambda i: (0, 0)),
            pl.BlockSpec((D, D), lambda i: (0, 0)),
        ],
        out_specs=[
            pl.BlockSpec((br, D), lambda i: (i, 0)),
            pl.BlockSpec((br, D), lambda i: (i, 0)),
        ],
        out_shape=[jax.ShapeDtypeStruct((N, D), jnp.float32)] * 2,
    )(x, wl, wr)


def _epi_body(pa_ref, pb_ref, b_ref, g_ref, be_ref, xp_ref, o_ref, *,
              use_gelu):
    s = jnp.concatenate(
        [pa_ref[0] + pa_ref[1], pb_ref[0] + pb_ref[1]], axis=-1) + b_ref[...]
    mu = jnp.mean(s, axis=-1, keepdims=True)
    var = jnp.mean(jnp.square(s - mu), axis=-1, keepdims=True)
    out = (s - mu) * lax.rsqrt(var + 1e-5) * g_ref[...] + be_ref[...]
    if use_gelu:
        out = jax.nn.gelu(out)
    o_ref[...] = xp_ref[...] + out


def _epi(pa, pb, b, gamma, beta, x_prev, use_gelu):
    br = 400
    hd = D // 2
    return pl.pallas_call(
        functools.partial(_epi_body, use_gelu=use_gelu),
        grid=(N // br,),
        in_specs=[
            pl.BlockSpec((2, br, hd), lambda i: (0, i, 0)),
            pl.BlockSpec((2, br, hd), lambda i: (0, i, 0)),
            pl.BlockSpec((1, D), lambda i: (0, 0)),
            pl.BlockSpec((1, D), lambda i: (0, 0)),
            pl.BlockSpec((1, D), lambda i: (0, 0)),
            pl.BlockSpec((br, D), lambda i: (i, 0)),
        ],
        out_specs=pl.BlockSpec((br, D), lambda i: (i, 0)),
        out_shape=jax.ShapeDtypeStruct((N, D), jnp.float32),
    )(pa, pb, b.reshape(1, D), gamma.reshape(1, D), beta.reshape(1, D),
      x_prev)


def _make_pass1(h):
    c = D // h
    k = 80
    g_per = k // 16
    nchunks = EPW // k

    @functools.partial(
        pl.kernel,
        out_type=[
            jax.ShapeDtypeStruct((E * h,), jnp.float32),
            jax.ShapeDtypeStruct((NC, N2, 16), jnp.float32),
        ],
        mesh=_MESH,
        compiler_params=pltpu.CompilerParams(
            needs_layout_passes=False, use_tc_tiling_on_sc=False),
        scratch_types=[
            pltpu.VMEM((k,), jnp.int32),
            pltpu.VMEM((k,), jnp.int32),
            pltpu.VMEM((k,), jnp.float32),
            pltpu.VMEM((k, D), jnp.float32),
            pltpu.VMEM((k, D), jnp.float32),
            pltpu.VMEM((k, 16), jnp.float32),
            pltpu.VMEM((k * h,), jnp.float32),
            pltpu.VMEM((2 * D,), jnp.float32),
            pltpu.VMEM_SHARED((N2, 16), jnp.float32),
            pltpu.SemaphoreType.DMA,
            pltpu.SemaphoreType.DMA,
        ],
    )
    def pass1(src_hbm, dst_hbm, a_hbm, xl_hbm, xr_hbm, wea_hbm,
              ex_hbm, den_hbm,
              srcv, dstv, av, xlg, xrg, exb, exbf, wea, shden, sem1, sem2):
        scid = lax.axis_index("c")
        sid = lax.axis_index("s")
        wid = sid * NC + scid
        lanes = _iota16()
        zero16 = jnp.zeros((16,), jnp.float32)

        pltpu.sync_copy(wea_hbm, wea)

        def zb(i, _):
            plsc.store_scatter(exb, [jnp.full((16,), i, jnp.int32), lanes],
                               zero16)
            return _
        lax.fori_loop(0, k, zb, 0)
        nz = ROWS_PT // k
        for z in range(nz):
            pltpu.sync_copy(
                exb, shden.at[pl.ds(sid * ROWS_PT + z * k, k)])
        rem = ROWS_PT - nz * k
        if rem:
            pltpu.sync_copy(
                exb.at[pl.ds(0, rem)],
                shden.at[pl.ds(sid * ROWS_PT + nz * k, rem)])
        plsc.subcore_barrier()

        def chunk(t, _):
            base = wid * EPW + t * k
            pltpu.sync_copy(src_hbm.at[pl.ds(base, k)], srcv)
            pltpu.sync_copy(dst_hbm.at[pl.ds(base, k)], dstv)
            pltpu.sync_copy(a_hbm.at[pl.ds(base, k)], av)
            cp1 = pltpu.async_copy(xl_hbm.at[srcv], xlg, sem1)
            cp2 = pltpu.async_copy(xr_hbm.at[dstv], xrg, sem2)
            cp1.wait()
            cp2.wait()

            def group(g, _):
                rowidx = g * 16 + lanes
                a16 = av[pl.ds(g * 16, 16)]
                for hh in range(h):
                    def jbody(j, acc):
                        cj = jnp.full((16,), j, jnp.int32)
                        xlv = plsc.load_gather(xlg, [rowidx, cj])
                        xrv = plsc.load_gather(xrg, [rowidx, cj])
                        wev = plsc.load_gather(wea, [cj])
                        atv = plsc.load_gather(wea, [cj + D])
                        m = xlv + xrv + a16 * wev
                        m = jnp.maximum(m, 0.2 * m)
                        return acc + m * atv
                    acc = lax.fori_loop(hh * c, (hh + 1) * c, jbody,
                                        jnp.zeros((16,), jnp.float32))
                    exval = jnp.exp(acc)
                    plsc.store_scatter(
                        exb, [rowidx, jnp.full((16,), hh, jnp.int32)], exval)
                    plsc.store_scatter(exbf, [rowidx * h + hh], exval)
                return _
            lax.fori_loop(0, g_per, group, 0)

            pltpu.sync_copy(exbf, ex_hbm.at[pl.ds(base * h, k * h)])
            pltpu.sync_copy(exb, shden.at[dstv], add=True)
            return _
        lax.fori_loop(0, nchunks, chunk, 0)

        plsc.subcore_barrier()
        pltpu.sync_copy(
            shden.at[pl.ds(sid * ROWS_PT, ROWS_PT)],
            den_hbm.at[scid, pl.ds(sid * ROWS_PT, ROWS_PT)])

    return pass1


def _make_pass2(h):
    c = D // h
    k = 80
    g_per = k // 16
    nchunks = EPW // k
    cb = ROWS_PT
    HD = D // 2

    @functools.partial(
        pl.kernel,
        out_type=[
            jax.ShapeDtypeStruct((NC, N2, HD), jnp.float32),
            jax.ShapeDtypeStruct((NC, N2, HD), jnp.float32),
        ],
        mesh=_MESH,
        compiler_params=pltpu.CompilerParams(
            needs_layout_passes=False, use_tc_tiling_on_sc=False),
        scratch_types=[
            pltpu.VMEM((k,), jnp.int32),
            pltpu.VMEM((k,), jnp.int32),
            pltpu.VMEM((k, HD), jnp.float32),
            pltpu.VMEM((k * h,), jnp.float32),
            pltpu.VMEM((k, 16), jnp.float32),
            pltpu.VMEM((160, 16), jnp.float32),
            pltpu.VMEM((160, 16), jnp.float32),
            pltpu.VMEM((h * 16,), jnp.float32),
            pltpu.VMEM_SHARED((N2, 16), jnp.float32),
            pltpu.VMEM_SHARED((N2, HD), jnp.float32),
            pltpu.SemaphoreType.DMA,
            pltpu.SemaphoreType.DMA,
        ],
    )
    def pass2(src_hbm, dst_hbm, xla_hbm, xlb_hbm, ex_hbm, den_hbm,
              outa_hbm, outb_hbm,
              srcv, dstv, xlg, exbf, dvb, db0, db1, abuf, shinv, shout,
              sem1, sem2):
        scid = lax.axis_index("c")
        sid = lax.axis_index("s")
        wid = sid * NC + scid
        lanes = _iota16()
        zero16 = jnp.zeros((16,), jnp.float32)
        colv = [lanes + v * 16 for v in range(HD // 16)]

        for q in range(cb // 160):
            pltpu.sync_copy(den_hbm.at[0, pl.ds(sid * cb + q * 160, 160)],
                            db0)
            pltpu.sync_copy(den_hbm.at[1, pl.ds(sid * cb + q * 160, 160)],
                            db1)

            def dstep(r, _):
                rv = jnp.full((16,), r, jnp.int32)
                v = (plsc.load_gather(db0, [rv, lanes])
                     + plsc.load_gather(db1, [rv, lanes]))
                plsc.store_scatter(db0, [rv, lanes], 1.0 / (v + 1e-16))
                return _
            lax.fori_loop(0, 160, dstep, 0)
            pltpu.sync_copy(db0, shinv.at[pl.ds(sid * cb + q * 160, 160)])

        def zero_xlg(i, _):
            plsc.store_scatter(
                xlg, [jnp.full((16,), i // (HD // 16), jnp.int32),
                      (i % (HD // 16)) * 16 + lanes], zero16)
            return _

        def phase(xl_hbm, out_hbm, feat_base):
            lax.fori_loop(0, k * (HD // 16), zero_xlg, 0)
            for z in range(ROWS_PT // k):
                pltpu.sync_copy(
                    xlg, shout.at[pl.ds(sid * ROWS_PT + z * k, k)])
            plsc.subcore_barrier()

            def chunk(t, _):
                base = wid * EPW + t * k
                pltpu.sync_copy(src_hbm.at[pl.ds(base, k)], srcv)
                pltpu.sync_copy(dst_hbm.at[pl.ds(base, k)], dstv)
                pltpu.sync_copy(ex_hbm.at[pl.ds(base * h, k * h)], exbf)
                cp1 = pltpu.async_copy(xl_hbm.at[srcv], xlg, sem1)
                cp2 = pltpu.async_copy(shinv.at[dstv], dvb, sem2)
                cp1.wait()
                cp2.wait()

                def group(g, _):
                    rowidx = g * 16 + lanes
                    alph = []
                    for hh in range(h):
                        hv = jnp.full((16,), hh, jnp.int32)
                        exv = plsc.load_gather(exbf, [rowidx * h + hh])
                        dv = plsc.load_gather(dvb, [rowidx, hv])
                        alph.append(exv * dv)
                    for i in range(16):
                        row = jnp.full((16,), g * 16 + i, jnp.int32)
                        iv = jnp.full((16,), i, jnp.int32)
                        ab = [jnp.take_along_axis(al, iv, axis=0)
                              for al in alph]
                        for v in range(HD // 16):
                            xv = plsc.load_gather(xlg, [row, colv[v]])
                            plsc.store_scatter(
                                xlg, [row, colv[v]],
                                xv * ab[(feat_base + v * 16) // c])
                    return _
                lax.fori_loop(0, g_per, group, 0)

                pltpu.sync_copy(xlg, shout.at[dstv], add=True)
                return _
            lax.fori_loop(0, nchunks, chunk, 0)

            plsc.subcore_barrier()
            pltpu.sync_copy(
                shout.at[pl.ds(sid * ROWS_PT, ROWS_PT)],
                out_hbm.at[scid, pl.ds(sid * ROWS_PT, ROWS_PT)])
            plsc.subcore_barrier()

        phase(xla_hbm, outa_hbm, 0)
        phase(xlb_hbm, outb_hbm, HD)

    return pass2


_PASS1 = {h: _make_pass1(h) for h in (4, 1)}
_PASS2 = {h: _make_pass2(h) for h in (4, 1)}


def kernel(x, edge_index, edge_attr, batch, params):
    src = edge_index[0].astype(jnp.int32)
    dst = edge_index[1].astype(jnp.int32)
    a = edge_attr[:, 0].astype(jnp.float32)
    for i, p in enumerate(params):
        h = p["att"].shape[0]
        xl, xr = _mm(x, p["Wl"], p["Wr"])
        wea = jnp.concatenate([p["We"][0], p["att"].reshape(-1)])
        ex, den = _PASS1[h](src, dst, a, xl, xr, wea)
        xla = xl[:, :D // 2]
        xlb = xl[:, D // 2:]
        pa, pb = _PASS2[h](src, dst, xla, xlb, ex, den)
        x = _epi(pa, pb, p["b"], p["gamma"], p["beta"], x, use_gelu=i < 2)
    return x[None]


# speedup vs baseline: 12.9985x; 1.3952x over previous
import functools

import jax
import jax.numpy as jnp
from jax import lax
from jax.experimental import pallas as pl
from jax.experimental.pallas import tpu as pltpu
from jax.experimental.pallas import tpu_sc as plsc

N = 10000
E = 320000
D = 128
NC = 2
NS = 16
NW = NC * NS
EPW = E // NW
N2 = 10240
ROWS_PT = N2 // NS

_MESH = plsc.VectorSubcoreMesh(core_axis_name="c", subcore_axis_name="s")


def _iota16():
    return lax.broadcasted_iota(jnp.int32, (16,), 0)


def _mm_body(x_ref, wl_ref, wr_ref, xl_ref, xr_ref):
    x = x_ref[...]
    xl_ref[...] = jnp.dot(x, wl_ref[...], preferred_element_type=jnp.float32)
    xr_ref[...] = jnp.dot(x, wr_ref[...], preferred_element_type=jnp.float32)


def _mm(x, wl, wr):
    br = 400
    return pl.pallas_call(
        _mm_body,
        grid=(N // br,),
        in_specs=[
            pl.BlockSpec((br, D), lambda i: (i, 0)),
            pl.BlockSpec((D, D), lambda i: (0, 0)),
            pl.BlockSpec((D, D), lambda i: (0, 0)),
        ],
        out_specs=[
            pl.BlockSpec((br, D), lambda i: (i, 0)),
            pl.BlockSpec((br, D), lambda i: (i, 0)),
        ],
        out_shape=[jax.ShapeDtypeStruct((N, D), jnp.float32)] * 2,
    )(x, wl, wr)


def _epi_body(pa_ref, pb_ref, b_ref, g_ref, be_ref, xp_ref, o_ref, *,
              use_gelu):
    s = jnp.concatenate(
        [pa_ref[0] + pa_ref[1], pb_ref[0] + pb_ref[1]], axis=-1) + b_ref[...]
    mu = jnp.mean(s, axis=-1, keepdims=True)
    var = jnp.mean(jnp.square(s - mu), axis=-1, keepdims=True)
    out = (s - mu) * lax.rsqrt(var + 1e-5) * g_ref[...] + be_ref[...]
    if use_gelu:
        out = jax.nn.gelu(out)
    o_ref[...] = xp_ref[...] + out


def _epi(pa, pb, b, gamma, beta, x_prev, use_gelu):
    br = 400
    hd = D // 2
    return pl.pallas_call(
        functools.partial(_epi_body, use_gelu=use_gelu),
        grid=(N // br,),
        in_specs=[
            pl.BlockSpec((2, br, hd), lambda i: (0, i, 0)),
            pl.BlockSpec((2, br, hd), lambda i: (0, i, 0)),
            pl.BlockSpec((1, D), lambda i: (0, 0)),
            pl.BlockSpec((1, D), lambda i: (0, 0)),
            pl.BlockSpec((1, D), lambda i: (0, 0)),
            pl.BlockSpec((br, D), lambda i: (i, 0)),
        ],
        out_specs=pl.BlockSpec((br, D), lambda i: (i, 0)),
        out_shape=jax.ShapeDtypeStruct((N, D), jnp.float32),
    )(pa, pb, b.reshape(1, D), gamma.reshape(1, D), beta.reshape(1, D),
      x_prev)


def _inv_body(d_ref, o_ref):
    o_ref[...] = 1.0 / (d_ref[0] + d_ref[1] + 1e-16)


def _inv(den):
    r = N2 * 16 // 128
    d = den.reshape(NC, r, 128)
    out = pl.pallas_call(
        _inv_body,
        in_specs=[pl.BlockSpec((NC, r, 128), lambda: (0, 0, 0))],
        out_specs=pl.BlockSpec((r, 128), lambda: (0, 0)),
        out_shape=jax.ShapeDtypeStruct((r, 128), jnp.float32),
    )(d)
    return out.reshape(N2, 16)


def _make_pass1(h):
    c = D // h
    k = 80
    g_per = k // 16
    nchunks = EPW // k

    @functools.partial(
        pl.kernel,
        out_type=[
            jax.ShapeDtypeStruct((E * h,), jnp.float32),
            jax.ShapeDtypeStruct((NC, N2, 16), jnp.float32),
        ],
        mesh=_MESH,
        compiler_params=pltpu.CompilerParams(
            needs_layout_passes=False, use_tc_tiling_on_sc=False),
        scratch_types=[
            pltpu.VMEM((nchunks, k), jnp.int32),
            pltpu.VMEM((nchunks, k), jnp.int32),
            pltpu.VMEM((nchunks, k), jnp.float32),
            pltpu.VMEM((k, D), jnp.float32),
            pltpu.VMEM((k, D), jnp.float32),
            pltpu.VMEM((k, D), jnp.float32),
            pltpu.VMEM((k, D), jnp.float32),
            pltpu.VMEM((k, 16), jnp.float32),
            pltpu.VMEM((k * h,), jnp.float32),
            pltpu.VMEM((2 * D,), jnp.float32),
            pltpu.VMEM_SHARED((N2, 16), jnp.float32),
            pltpu.SemaphoreType.DMA,
            pltpu.SemaphoreType.DMA,
            pltpu.SemaphoreType.DMA,
            pltpu.SemaphoreType.DMA,
        ],
    )
    def pass1(src_hbm, dst_hbm, a_hbm, xl_hbm, xr_hbm, wea_hbm,
              ex_hbm, den_hbm,
              srcall, dstall, aall, xlg0, xlg1, xrg0, xrg1, exb, exbf, wea,
              shden, sl0, sl1, sr0, sr1):
        scid = lax.axis_index("c")
        sid = lax.axis_index("s")
        wid = sid * NC + scid
        lanes = _iota16()
        zero16 = jnp.zeros((16,), jnp.float32)
        bufs = [(xlg0, xrg0, sl0, sr0), (xlg1, xrg1, sl1, sr1)]

        pltpu.sync_copy(wea_hbm, wea)
        pltpu.sync_copy(src_hbm.at[pl.ds(wid * nchunks, nchunks)], srcall)
        pltpu.sync_copy(dst_hbm.at[pl.ds(wid * nchunks, nchunks)], dstall)
        pltpu.sync_copy(a_hbm.at[pl.ds(wid * nchunks, nchunks)], aall)

        def zb(i, _):
            plsc.store_scatter(exb, [jnp.full((16,), i, jnp.int32), lanes],
                               zero16)
            return _
        lax.fori_loop(0, k, zb, 0)
        for z in range(ROWS_PT // k):
            pltpu.sync_copy(
                exb, shden.at[pl.ds(sid * ROWS_PT + z * k, k)])
        plsc.subcore_barrier()

        def issue(t, b):
            xlg, xrg, sl, sr = bufs[b]
            pltpu.async_copy(xl_hbm.at[srcall.at[t]], xlg, sl)
            pltpu.async_copy(xr_hbm.at[dstall.at[t]], xrg, sr)

        def process(t, b, do_issue):
            xlg, xrg, sl, sr = bufs[b]
            pltpu.make_async_copy(xl_hbm.at[srcall.at[t]], xlg, sl).wait()
            pltpu.make_async_copy(xr_hbm.at[dstall.at[t]], xrg, sr).wait()

            def group(g, _):
                rowidx = g * 16 + lanes
                a16 = plsc.load_gather(
                    aall, [jnp.full((16,), t, jnp.int32), rowidx])
                for hh in range(h):
                    def jbody(j, acc):
                        cj = jnp.full((16,), j, jnp.int32)
                        xlv = plsc.load_gather(xlg, [rowidx, cj])
                        xrv = plsc.load_gather(xrg, [rowidx, cj])
                        wev = plsc.load_gather(wea, [cj])
                        atv = plsc.load_gather(wea, [cj + D])
                        m = xlv + xrv + a16 * wev
                        m = jnp.maximum(m, 0.2 * m)
                        return acc + m * atv
                    acc = lax.fori_loop(hh * c, (hh + 1) * c, jbody,
                                        jnp.zeros((16,), jnp.float32))
                    exval = jnp.exp(acc)
                    plsc.store_scatter(
                        exb, [rowidx, jnp.full((16,), hh, jnp.int32)], exval)
                    plsc.store_scatter(exbf, [rowidx * h + hh], exval)
                return _
            lax.fori_loop(0, g_per, group, 0)

            if do_issue:
                @pl.when(t + 2 < nchunks)
                def _():
                    issue(t + 2, b)

            base = wid * EPW + t * k
            pltpu.sync_copy(exbf, ex_hbm.at[pl.ds(base * h, k * h)])
            pltpu.sync_copy(exb, shden.at[dstall.at[t]], add=True)

        issue(0, 0)
        issue(1, 1)

        def pair(t2, _):
            process(t2 * 2, 0, True)
            process(t2 * 2 + 1, 1, True)
            return _
        lax.fori_loop(0, nchunks // 2, pair, 0)
        process(nchunks - 1, (nchunks - 1) % 2, False)

        plsc.subcore_barrier()
        pltpu.sync_copy(
            shden.at[pl.ds(sid * ROWS_PT, ROWS_PT)],
            den_hbm.at[scid, pl.ds(sid * ROWS_PT, ROWS_PT)])

    return pass1


def _make_pass2(h):
    c = D // h
    k = 80
    g_per = k // 16
    nchunks = EPW // k
    HD = D // 2

    @functools.partial(
        pl.kernel,
        out_type=[
            jax.ShapeDtypeStruct((NC, N2, HD), jnp.float32),
            jax.ShapeDtypeStruct((NC, N2, HD), jnp.float32),
        ],
        mesh=_MESH,
        compiler_params=pltpu.CompilerParams(
            needs_layout_passes=False, use_tc_tiling_on_sc=False),
        scratch_types=[
            pltpu.VMEM((nchunks, k), jnp.int32),
            pltpu.VMEM((nchunks, k), jnp.int32),
            pltpu.VMEM((k, HD), jnp.float32),
            pltpu.VMEM((k, HD), jnp.float32),
            pltpu.VMEM((k, 16), jnp.float32),
            pltpu.VMEM((k, 16), jnp.float32),
            pltpu.VMEM((k * h,), jnp.float32),
            pltpu.VMEM((k * h,), jnp.float32),
            pltpu.VMEM_SHARED((N2, HD), jnp.float32),
            pltpu.SemaphoreType.DMA,
            pltpu.SemaphoreType.DMA,
            pltpu.SemaphoreType.DMA,
            pltpu.SemaphoreType.DMA,
            pltpu.SemaphoreType.DMA,
            pltpu.SemaphoreType.DMA,
        ],
    )
    def pass2(src_hbm, dst_hbm, xla_hbm, xlb_hbm, ex_hbm, inv_hbm,
              outa_hbm, outb_hbm,
              srcall, dstall, xg0, xg1, dv0, dv1, ex0, ex1, shout,
              sa0, sa1, sb0, sb1, sc0, sc1):
        scid = lax.axis_index("c")
        sid = lax.axis_index("s")
        wid = sid * NC + scid
        lanes = _iota16()
        zero16 = jnp.zeros((16,), jnp.float32)
        colv = [lanes + v * 16 for v in range(HD // 16)]
        bufs = [(xg0, dv0, ex0, sa0, sb0, sc0), (xg1, dv1, ex1, sa1, sb1, sc1)]

        pltpu.sync_copy(src_hbm.at[pl.ds(wid * nchunks, nchunks)], srcall)
        pltpu.sync_copy(dst_hbm.at[pl.ds(wid * nchunks, nchunks)], dstall)

        def zero_xg0(i, _):
            plsc.store_scatter(
                xg0, [jnp.full((16,), i // (HD // 16), jnp.int32),
                      (i % (HD // 16)) * 16 + lanes], zero16)
            return _

        def phase(xl_hbm, out_hbm, feat_base):
            lax.fori_loop(0, k * (HD // 16), zero_xg0, 0)
            for z in range(ROWS_PT // k):
                pltpu.sync_copy(
                    xg0, shout.at[pl.ds(sid * ROWS_PT + z * k, k)])
            plsc.subcore_barrier()

            def issue(t, b):
                xg, dv, exv, sa, sb, sc = bufs[b]
                base = wid * EPW + t * k
                pltpu.async_copy(xl_hbm.at[srcall.at[t]], xg, sa)
                pltpu.async_copy(inv_hbm.at[dstall.at[t]], dv, sb)
                pltpu.async_copy(ex_hbm.at[pl.ds(base * h, k * h)], exv, sc)

            def process(t, b, do_issue):
                xg, dv, exv, sa, sb, sc = bufs[b]
                base = wid * EPW + t * k
                pltpu.make_async_copy(xl_hbm.at[srcall.at[t]], xg, sa).wait()
                pltpu.make_async_copy(inv_hbm.at[dstall.at[t]], dv, sb).wait()
                pltpu.make_async_copy(
                    ex_hbm.at[pl.ds(base * h, k * h)], exv, sc).wait()

                def group(g, _):
                    rowidx = g * 16 + lanes
                    alph = []
                    for hh in range(h):
                        hv = jnp.full((16,), hh, jnp.int32)
                        exl = plsc.load_gather(exv, [rowidx * h + hh])
                        dvl = plsc.load_gather(dv, [rowidx, hv])
                        alph.append(exl * dvl)
                    for i in range(16):
                        row = jnp.full((16,), g * 16 + i, jnp.int32)
                        iv = jnp.full((16,), i, jnp.int32)
                        ab = [jnp.take_along_axis(al, iv, axis=0)
                              for al in alph]
                        for v in range(HD // 16):
                            xv = plsc.load_gather(xg, [row, colv[v]])
                            plsc.store_scatter(
                                xg, [row, colv[v]],
                                xv * ab[(feat_base + v * 16) // c])
                    return _
                lax.fori_loop(0, g_per, group, 0)

                if do_issue:
                    @pl.when(t + 2 < nchunks)
                    def _():
                        issue(t + 2, b)

                pltpu.sync_copy(xg, shout.at[dstall.at[t]], add=True)

            issue(0, 0)
            issue(1, 1)

            def pair(t2, _):
                process(t2 * 2, 0, True)
                process(t2 * 2 + 1, 1, True)
                return _
            lax.fori_loop(0, nchunks // 2, pair, 0)
            process(nchunks - 1, (nchunks - 1) % 2, False)

            plsc.subcore_barrier()
            pltpu.sync_copy(
                shout.at[pl.ds(sid * ROWS_PT, ROWS_PT)],
                out_hbm.at[scid, pl.ds(sid * ROWS_PT, ROWS_PT)])
            plsc.subcore_barrier()

        phase(xla_hbm, outa_hbm, 0)
        phase(xlb_hbm, outb_hbm, HD)

    return pass2


_PASS1 = {h: _make_pass1(h) for h in (4, 1)}
_PASS2 = {h: _make_pass2(h) for h in (4, 1)}


def kernel(x, edge_index, edge_attr, batch, params):
    k = 80
    src = edge_index[0].astype(jnp.int32).reshape(E // k, k)
    dst = edge_index[1].astype(jnp.int32).reshape(E // k, k)
    a = edge_attr[:, 0].astype(jnp.float32).reshape(E // k, k)
    for i, p in enumerate(params):
        h = p["att"].shape[0]
        xl, xr = _mm(x, p["Wl"], p["Wr"])
        wea = jnp.concatenate([p["We"][0], p["att"].reshape(-1)])
        ex, den = _PASS1[h](src, dst, a, xl, xr, wea)
        xla = xl[:, :D // 2]
        xlb = xl[:, D // 2:]
        pa, pb = _PASS2[h](src, dst, xla, xlb, ex, _inv(den))
        x = _epi(pa, pb, p["b"], p["gamma"], p["beta"], x, use_gelu=i < 2)
    return x[None]


# speedup vs baseline: 13.5389x; 1.0416x over previous
import functools

import jax
import jax.numpy as jnp
from jax import lax
from jax.experimental import pallas as pl
from jax.experimental.pallas import tpu as pltpu
from jax.experimental.pallas import tpu_sc as plsc

N = 10000
E = 320000
D = 128
NC = 2
NS = 16
NW = NC * NS
EPW = E // NW
N2 = 10240
ROWS_PT = N2 // NS

_MESH = plsc.VectorSubcoreMesh(core_axis_name="c", subcore_axis_name="s")


def _iota16():
    return lax.broadcasted_iota(jnp.int32, (16,), 0)


def _mm_body(x_ref, wl_ref, wr_ref, xl_ref, xr_ref):
    x = x_ref[...]
    xl_ref[...] = jnp.dot(x, wl_ref[...], preferred_element_type=jnp.float32)
    xr_ref[...] = jnp.dot(x, wr_ref[...], preferred_element_type=jnp.float32)


def _mm(x, wl, wr):
    br = 400
    return pl.pallas_call(
        _mm_body,
        grid=(N // br,),
        in_specs=[
            pl.BlockSpec((br, D), lambda i: (i, 0)),
            pl.BlockSpec((D, D), lambda i: (0, 0)),
            pl.BlockSpec((D, D), lambda i: (0, 0)),
        ],
        out_specs=[
            pl.BlockSpec((br, D), lambda i: (i, 0)),
            pl.BlockSpec((br, D), lambda i: (i, 0)),
        ],
        out_shape=[jax.ShapeDtypeStruct((N, D), jnp.float32)] * 2,
    )(x, wl, wr)


def _epi_body(pa_ref, pb_ref, b_ref, g_ref, be_ref, xp_ref, o_ref, *,
              use_gelu):
    s = jnp.concatenate(
        [pa_ref[0] + pa_ref[1], pb_ref[0] + pb_ref[1]], axis=-1) + b_ref[...]
    mu = jnp.mean(s, axis=-1, keepdims=True)
    var = jnp.mean(jnp.square(s - mu), axis=-1, keepdims=True)
    out = (s - mu) * lax.rsqrt(var + 1e-5) * g_ref[...] + be_ref[...]
    if use_gelu:
        out = jax.nn.gelu(out)
    o_ref[...] = xp_ref[...] + out


def _epi(pa, pb, b, gamma, beta, x_prev, use_gelu):
    br = 400
    hd = D // 2
    return pl.pallas_call(
        functools.partial(_epi_body, use_gelu=use_gelu),
        grid=(N // br,),
        in_specs=[
            pl.BlockSpec((2, br, hd), lambda i: (0, i, 0)),
            pl.BlockSpec((2, br, hd), lambda i: (0, i, 0)),
            pl.BlockSpec((1, D), lambda i: (0, 0)),
            pl.BlockSpec((1, D), lambda i: (0, 0)),
            pl.BlockSpec((1, D), lambda i: (0, 0)),
            pl.BlockSpec((br, D), lambda i: (i, 0)),
        ],
        out_specs=pl.BlockSpec((br, D), lambda i: (i, 0)),
        out_shape=jax.ShapeDtypeStruct((N, D), jnp.float32),
    )(pa, pb, b.reshape(1, D), gamma.reshape(1, D), beta.reshape(1, D),
      x_prev)


def _inv_body(d_ref, o_ref):
    o_ref[...] = 1.0 / (d_ref[0] + d_ref[1] + 1e-16)


def _inv(den):
    r = N2 * 16 // 128
    d = den.reshape(NC, r, 128)
    out = pl.pallas_call(
        _inv_body,
        in_specs=[pl.BlockSpec((NC, r, 128), lambda: (0, 0, 0))],
        out_specs=pl.BlockSpec((r, 128), lambda: (0, 0)),
        out_shape=jax.ShapeDtypeStruct((r, 128), jnp.float32),
    )(d)
    return out.reshape(N2, 16)


def _make_pass1(h):
    c = D // h
    k = 80
    g_per = k // 16
    nchunks = EPW // k

    @functools.partial(
        pl.kernel,
        out_type=[
            jax.ShapeDtypeStruct((E * h,), jnp.float32),
            jax.ShapeDtypeStruct((NC, N2, 16), jnp.float32),
        ],
        mesh=_MESH,
        compiler_params=pltpu.CompilerParams(
            needs_layout_passes=False, use_tc_tiling_on_sc=False),
        scratch_types=[
            pltpu.VMEM((nchunks, k), jnp.int32),
            pltpu.VMEM((nchunks, k), jnp.int32),
            pltpu.VMEM((nchunks, k), jnp.float32),
            pltpu.VMEM((k, D), jnp.float32),
            pltpu.VMEM((k, D), jnp.float32),
            pltpu.VMEM((k, D), jnp.float32),
            pltpu.VMEM((k, D), jnp.float32),
            pltpu.VMEM((k, 16), jnp.float32),
            pltpu.VMEM((k * h,), jnp.float32),
            pltpu.VMEM((2 * D,), jnp.float32),
            pltpu.VMEM_SHARED((N2, 16), jnp.float32),
            pltpu.SemaphoreType.DMA,
            pltpu.SemaphoreType.DMA,
            pltpu.SemaphoreType.DMA,
            pltpu.SemaphoreType.DMA,
        ],
    )
    def pass1(src_hbm, dst_hbm, a_hbm, xl_hbm, xr_hbm, wea_hbm,
              ex_hbm, den_hbm,
              srcall, dstall, aall, xlg0, xlg1, xrg0, xrg1, exb, exbf, wea,
              shden, sl0, sl1, sr0, sr1):
        scid = lax.axis_index("c")
        sid = lax.axis_index("s")
        wid = sid * NC + scid
        lanes = _iota16()
        zero16 = jnp.zeros((16,), jnp.float32)
        bufs = [(xlg0, xrg0, sl0, sr0), (xlg1, xrg1, sl1, sr1)]

        pltpu.sync_copy(wea_hbm, wea)
        pltpu.sync_copy(src_hbm.at[pl.ds(wid * nchunks, nchunks)], srcall)
        pltpu.sync_copy(dst_hbm.at[pl.ds(wid * nchunks, nchunks)], dstall)
        pltpu.sync_copy(a_hbm.at[pl.ds(wid * nchunks, nchunks)], aall)

        def zb(i, _):
            plsc.store_scatter(exb, [jnp.full((16,), i, jnp.int32), lanes],
                               zero16)
            return _
        lax.fori_loop(0, k, zb, 0)
        for z in range(ROWS_PT // k):
            pltpu.sync_copy(
                exb, shden.at[pl.ds(sid * ROWS_PT + z * k, k)])
        plsc.subcore_barrier()

        def issue(t, b):
            xlg, xrg, sl, sr = bufs[b]
            pltpu.async_copy(xl_hbm.at[srcall.at[t]], xlg, sl)
            pltpu.async_copy(xr_hbm.at[dstall.at[t]], xrg, sr)

        def process(t, b, do_issue):
            xlg, xrg, sl, sr = bufs[b]
            pltpu.make_async_copy(xl_hbm.at[srcall.at[t]], xlg, sl).wait()
            pltpu.make_async_copy(xr_hbm.at[dstall.at[t]], xrg, sr).wait()

            rowg = [g * 16 + lanes for g in range(g_per)]
            a16g = [plsc.load_gather(
                aall, [jnp.full((16,), t, jnp.int32), rowg[g]])
                for g in range(g_per)]
            for hh in range(h):
                def jbody(j, accs):
                    cj = jnp.full((16,), j, jnp.int32)
                    wev = plsc.load_gather(wea, [cj])
                    atv = plsc.load_gather(wea, [cj + D])
                    out = []
                    for g in range(g_per):
                        xlv = plsc.load_gather(xlg, [rowg[g], cj])
                        xrv = plsc.load_gather(xrg, [rowg[g], cj])
                        m = xlv + xrv + a16g[g] * wev
                        m = jnp.maximum(m, 0.2 * m)
                        out.append(accs[g] + m * atv)
                    return tuple(out)
                accs = lax.fori_loop(
                    hh * c, (hh + 1) * c, jbody,
                    tuple(jnp.zeros((16,), jnp.float32)
                          for _ in range(g_per)))
                for g in range(g_per):
                    exval = jnp.exp(accs[g])
                    plsc.store_scatter(
                        exb, [rowg[g], jnp.full((16,), hh, jnp.int32)],
                        exval)
                    plsc.store_scatter(exbf, [rowg[g] * h + hh], exval)

            if do_issue:
                @pl.when(t + 2 < nchunks)
                def _():
                    issue(t + 2, b)

            base = wid * EPW + t * k
            pltpu.sync_copy(exbf, ex_hbm.at[pl.ds(base * h, k * h)])
            pltpu.sync_copy(exb, shden.at[dstall.at[t]], add=True)

        issue(0, 0)
        issue(1, 1)

        def pair(t2, _):
            process(t2 * 2, 0, True)
            process(t2 * 2 + 1, 1, True)
            return _
        lax.fori_loop(0, nchunks // 2, pair, 0)
        process(nchunks - 1, (nchunks - 1) % 2, False)

        plsc.subcore_barrier()
        pltpu.sync_copy(
            shden.at[pl.ds(sid * ROWS_PT, ROWS_PT)],
            den_hbm.at[scid, pl.ds(sid * ROWS_PT, ROWS_PT)])

    return pass1


def _make_pass2(h):
    c = D // h
    k = 80
    g_per = k // 16
    nchunks = EPW // k
    HD = D // 2

    @functools.partial(
        pl.kernel,
        out_type=[
            jax.ShapeDtypeStruct((NC, N2, HD), jnp.float32),
            jax.ShapeDtypeStruct((NC, N2, HD), jnp.float32),
        ],
        mesh=_MESH,
        compiler_params=pltpu.CompilerParams(
            needs_layout_passes=False, use_tc_tiling_on_sc=False),
        scratch_types=[
            pltpu.VMEM((nchunks, k), jnp.int32),
            pltpu.VMEM((nchunks, k), jnp.int32),
            pltpu.VMEM((k, HD), jnp.float32),
            pltpu.VMEM((k, HD), jnp.float32),
            pltpu.VMEM((k, 16), jnp.float32),
            pltpu.VMEM((k, 16), jnp.float32),
            pltpu.VMEM((k * h,), jnp.float32),
            pltpu.VMEM((k * h,), jnp.float32),
            pltpu.VMEM_SHARED((N2, HD), jnp.float32),
            pltpu.SemaphoreType.DMA,
            pltpu.SemaphoreType.DMA,
            pltpu.SemaphoreType.DMA,
            pltpu.SemaphoreType.DMA,
            pltpu.SemaphoreType.DMA,
            pltpu.SemaphoreType.DMA,
        ],
    )
    def pass2(src_hbm, dst_hbm, xla_hbm, xlb_hbm, ex_hbm, inv_hbm,
              outa_hbm, outb_hbm,
              srcall, dstall, xg0, xg1, dv0, dv1, ex0, ex1, shout,
              sa0, sa1, sb0, sb1, sc0, sc1):
        scid = lax.axis_index("c")
        sid = lax.axis_index("s")
        wid = sid * NC + scid
        lanes = _iota16()
        zero16 = jnp.zeros((16,), jnp.float32)
        colv = [lanes + v * 16 for v in range(HD // 16)]
        bufs = [(xg0, dv0, ex0, sa0, sb0, sc0), (xg1, dv1, ex1, sa1, sb1, sc1)]

        pltpu.sync_copy(src_hbm.at[pl.ds(wid * nchunks, nchunks)], srcall)
        pltpu.sync_copy(dst_hbm.at[pl.ds(wid * nchunks, nchunks)], dstall)

        def zero_xg0(i, _):
            plsc.store_scatter(
                xg0, [jnp.full((16,), i // (HD // 16), jnp.int32),
                      (i % (HD // 16)) * 16 + lanes], zero16)
            return _

        def phase(xl_hbm, out_hbm, feat_base):
            lax.fori_loop(0, k * (HD // 16), zero_xg0, 0)
            for z in range(ROWS_PT // k):
                pltpu.sync_copy(
                    xg0, shout.at[pl.ds(sid * ROWS_PT + z * k, k)])
            plsc.subcore_barrier()

            def issue(t, b):
                xg, dv, exv, sa, sb, sc = bufs[b]
                base = wid * EPW + t * k
                pltpu.async_copy(xl_hbm.at[srcall.at[t]], xg, sa)
                pltpu.async_copy(inv_hbm.at[dstall.at[t]], dv, sb)
                pltpu.async_copy(ex_hbm.at[pl.ds(base * h, k * h)], exv, sc)

            def process(t, b, do_issue):
                xg, dv, exv, sa, sb, sc = bufs[b]
                base = wid * EPW + t * k
                pltpu.make_async_copy(xl_hbm.at[srcall.at[t]], xg, sa).wait()
                pltpu.make_async_copy(inv_hbm.at[dstall.at[t]], dv, sb).wait()
                pltpu.make_async_copy(
                    ex_hbm.at[pl.ds(base * h, k * h)], exv, sc).wait()

                def group(g, _):
                    rowidx = g * 16 + lanes
                    alph = []
                    for hh in range(h):
                        hv = jnp.full((16,), hh, jnp.int32)
                        exl = plsc.load_gather(exv, [rowidx * h + hh])
                        dvl = plsc.load_gather(dv, [rowidx, hv])
                        alph.append(exl * dvl)
                    for i in range(16):
                        row = jnp.full((16,), g * 16 + i, jnp.int32)
                        iv = jnp.full((16,), i, jnp.int32)
                        ab = [jnp.take_along_axis(al, iv, axis=0)
                              for al in alph]
                        for v in range(HD // 16):
                            xv = plsc.load_gather(xg, [row, colv[v]])
                            plsc.store_scatter(
                                xg, [row, colv[v]],
                                xv * ab[(feat_base + v * 16) // c])
                    return _
                lax.fori_loop(0, g_per, group, 0)

                pltpu.sync_copy(xg, shout.at[dstall.at[t]], add=True)

                if do_issue:
                    @pl.when(t + 2 < nchunks)
                    def _():
                        issue(t + 2, b)

            issue(0, 0)
            issue(1, 1)

            def pair(t2, _):
                process(t2 * 2, 0, True)
                process(t2 * 2 + 1, 1, True)
                return _
            lax.fori_loop(0, nchunks // 2, pair, 0)
            process(nchunks - 1, (nchunks - 1) % 2, False)

            plsc.subcore_barrier()
            pltpu.sync_copy(
                shout.at[pl.ds(sid * ROWS_PT, ROWS_PT)],
                out_hbm.at[scid, pl.ds(sid * ROWS_PT, ROWS_PT)])
            plsc.subcore_barrier()

        phase(xla_hbm, outa_hbm, 0)
        phase(xlb_hbm, outb_hbm, HD)

    return pass2


_PASS1 = {h: _make_pass1(h) for h in (4, 1)}
_PASS2 = {h: _make_pass2(h) for h in (4, 1)}


def kernel(x, edge_index, edge_attr, batch, params):
    k = 80
    src = edge_index[0].astype(jnp.int32).reshape(E // k, k)
    dst = edge_index[1].astype(jnp.int32).reshape(E // k, k)
    a = edge_attr[:, 0].astype(jnp.float32).reshape(E // k, k)
    for i, p in enumerate(params):
        h = p["att"].shape[0]
        xl, xr = _mm(x, p["Wl"], p["Wr"])
        wea = jnp.concatenate([p["We"][0], p["att"].reshape(-1)])
        ex, den = _PASS1[h](src, dst, a, xl, xr, wea)
        xla = xl[:, :D // 2]
        xlb = xl[:, D // 2:]
        pa, pb = _PASS2[h](src, dst, xla, xlb, ex, _inv(den))
        x = _epi(pa, pb, p["b"], p["gamma"], p["beta"], x, use_gelu=i < 2)
    return x[None]
